# baseline (speedup 1.0000x reference)
import functools
import math

import jax
import jax.numpy as jnp
import numpy as np
from jax import lax
from jax.experimental import pallas as pl
from jax.experimental.pallas import tpu as pltpu
from jax.experimental.pallas import tpu_sc as plsc

_K = 10
_N = 10000
_E = 160000
_C = 128
_F2 = _C // 2

_NP = 10240
_CHUNK = 128
_NCHUNK = 81
_EPT = _CHUNK * _NCHUNK
_EPAD = _EPT * 16
_RPT = _NP // 16
_NB = 3


def _cheby_cols():
    k = _K
    nodes = [math.cos((k - j + 0.5) * math.pi / (k + 1)) for j in range(k + 1)]
    cols = []
    for xj in nodes:
        vals = [0.0] * (k + 1)
        for j in range(k + 1):
            if j == 0:
                vals[j] = 1.0
            elif j == 1:
                vals[j] = xj
            else:
                vals[j] = 2.0 * xj * vals[j - 1] - vals[j - 2]
        cols.append(vals)
    return np.array(cols, dtype=np.float32).T


_CV = _cheby_cols()


def _mlp_body(x_ref, w1_ref, b1_ref, w2a_ref, w2b_ref, b2_ref, o_ref):
    h = jnp.dot(x_ref[...], w1_ref[...], preferred_element_type=jnp.float32)
    h = jnp.maximum(h + b1_ref[...], 0.0)
    oa = jnp.dot(h, w2a_ref[...], preferred_element_type=jnp.float32)
    ob = jnp.dot(h, w2b_ref[...], preferred_element_type=jnp.float32)
    o_ref[0] = oa + b2_ref[0, 0:1, :]
    o_ref[1] = ob + b2_ref[0, 1:2, :]


def _mlp(x_pad, W1, b1, W2, b2):
    blk = 2048
    grid = (_NP // blk,)
    return pl.pallas_call(
        _mlp_body,
        grid=grid,
        in_specs=[
            pl.BlockSpec((blk, 256), lambda i: (i, 0)),
            pl.BlockSpec((256, 512), lambda i: (0, 0)),
            pl.BlockSpec((1, 512), lambda i: (0, 0)),
            pl.BlockSpec((512, _F2), lambda i: (0, 0)),
            pl.BlockSpec((512, _F2), lambda i: (0, 0)),
            pl.BlockSpec((1, 2, _F2), lambda i: (0, 0, 0)),
        ],
        out_specs=pl.BlockSpec((2, blk, _F2), lambda i: (0, i, 0)),
        out_shape=jax.ShapeDtypeStruct((2, _NP, _F2), jnp.float32),
    )(x_pad, W1, b1.reshape(1, 512), W2[:, :_F2], W2[:, _F2:],
      b2.reshape(1, 2, _F2))


def _spmm_sc_body(p_hbm, col_hbm, row_hbm, w_hbm, out_hbm,
                  colm, rowm, wm, rows_v, sbuf, accum_sh, sem_g, sem_s):
    c = lax.axis_index("c")
    s = lax.axis_index("s")

    pltpu.sync_copy(col_hbm.at[s], colm)
    pltpu.sync_copy(row_hbm.at[s], rowm)
    pltpu.sync_copy(w_hbm.at[s], wm)

    def _zero_body(r, carry):
        for f in range(_F2 // 16):
            for b in range(_NB):
                sbuf[b][r, pl.ds(f * 16, 16)] = jnp.zeros((16,), jnp.float32)
        return carry
    lax.fori_loop(0, _CHUNK, _zero_body, 0)
    for t in range(_RPT // _CHUNK):
        pltpu.sync_copy(sbuf[0], accum_sh.at[pl.ds(s * _RPT + t * _CHUNK, _CHUNK)])
    plsc.subcore_barrier()

    for b in range(_NB):
        pltpu.async_copy(p_hbm.at[c].at[colm.at[b]], rows_v[b], sem_g[b])
        pltpu.async_copy(sbuf[b], accum_sh.at[rowm.at[b]], sem_s[b], add=True)

    def _outer(jj, carry):
        for b in range(_NB):
            j = jj * _NB + b
            pltpu.make_async_copy(p_hbm.at[c].at[colm.at[j]], rows_v[b],
                                  sem_g[b]).wait()
            pltpu.make_async_copy(sbuf[b], accum_sh.at[rowm.at[j]],
                                  sem_s[b]).wait()

            base = j * _CHUNK

            def _mul_body(r, carry2):
                wv = plsc.load_gather(wm, [jnp.full((16,), base + r, jnp.int32)])
                for f in range(_F2 // 16):
                    sl = pl.ds(f * 16, 16)
                    sbuf[b][r, sl] = rows_v[b][r, sl] * wv
                return carry2
            lax.fori_loop(0, _CHUNK, _mul_body, 0, unroll=8)

            pltpu.async_copy(sbuf[b], accum_sh.at[rowm.at[j]], sem_s[b],
                             add=True)

            @pl.when(j + _NB < _NCHUNK)
            def _():
                pltpu.async_copy(p_hbm.at[c].at[colm.at[j + _NB]], rows_v[b],
                                 sem_g[b])
        return carry
    lax.fori_loop(0, _NCHUNK // _NB, _outer, 0)

    for b in range(_NB):
        pltpu.make_async_copy(sbuf[b], accum_sh.at[rowm.at[b]], sem_s[b]).wait()

    plsc.subcore_barrier()

    sl = pl.ds(s * _RPT, _RPT)
    pltpu.sync_copy(accum_sh.at[sl], out_hbm.at[c].at[sl])


def _make_spmm():
    mesh = plsc.VectorSubcoreMesh(core_axis_name="c", subcore_axis_name="s")
    return pl.kernel(
        _spmm_sc_body,
        out_type=jax.ShapeDtypeStruct((2, _NP, _F2), jnp.float32),
        mesh=mesh,
        scratch_types=[
            pltpu.VMEM((_NCHUNK, _CHUNK), jnp.int32),
            pltpu.VMEM((_NCHUNK, _CHUNK), jnp.int32),
            pltpu.VMEM((_EPT,), jnp.float32),
            [pltpu.VMEM((_CHUNK, _F2), jnp.float32) for _ in range(_NB)],
            [pltpu.VMEM((_CHUNK, _F2), jnp.float32) for _ in range(_NB)],
            pltpu.VMEM_SHARED((_NP, _F2), jnp.float32),
            [pltpu.SemaphoreType.DMA for _ in range(_NB)],
            [pltpu.SemaphoreType.DMA for _ in range(_NB)],
        ],
        compiler_params=pltpu.CompilerParams(use_tc_tiling_on_sc=False,
                                             needs_layout_passes=False),
    )


def _combine_body(first, t_ref, prev_ref, y_ref, fp_ref, p_ref, yo_ref):
    if first:
        p = t_ref[...]
        yo_ref[...] = fp_ref[0, 0] * prev_ref[...] + fp_ref[0, 1] * p
    else:
        p = 2.0 * t_ref[...] - prev_ref[...]
        yo_ref[...] = y_ref[...] + fp_ref[0, 0] * p
    p_ref[...] = p


def _combine(first, t, prev, y, fp_sc):
    blk = 2048
    grid = (_NP // blk,)
    spec = pl.BlockSpec((2, blk, _F2), lambda i: (0, i, 0))
    return pl.pallas_call(
        functools.partial(_combine_body, first),
        grid=grid,
        in_specs=[spec, spec, spec, pl.BlockSpec(memory_space=pltpu.SMEM)],
        out_specs=[spec, spec],
        out_shape=[
            jax.ShapeDtypeStruct((2, _NP, _F2), jnp.float32),
            jax.ShapeDtypeStruct((2, _NP, _F2), jnp.float32),
        ],
    )(t, prev, y, fp_sc)


def _logsoftmax_body(y_ref, o_ref):
    y = jnp.concatenate([y_ref[0], y_ref[1]], axis=1)
    m = jnp.max(y, axis=1, keepdims=True)
    e = jnp.exp(y - m)
    s = jnp.sum(e, axis=1, keepdims=True)
    o_ref[...] = y - m - jnp.log(s)


def _logsoftmax(y):
    blk = 2000
    return pl.pallas_call(
        _logsoftmax_body,
        grid=(_N // blk,),
        in_specs=[pl.BlockSpec((2, blk, _F2), lambda i: (0, i, 0))],
        out_specs=pl.BlockSpec((blk, _C), lambda i: (i, 0)),
        out_shape=jax.ShapeDtypeStruct((_N, _C), jnp.float32),
    )(y)


def kernel(x, edge_index, edge_weight, W1, b1, W2, b2, filter_param):
    fp = jnp.maximum(filter_param, 0.0)
    fp = jnp.asarray(_CV) @ fp
    fp = 2.0 * fp / (_K + 1)
    fp = fp.at[0].set(fp[0] / 2.0)
    signs = jnp.asarray(np.where(np.arange(_K + 1) % 2 == 0, 1.0, -1.0)
                        .astype(np.float32))
    fp = fp[:, 0] * signs

    pad = _EPAD - _E
    col = jnp.concatenate([edge_index[1].astype(jnp.int32),
                           jnp.zeros((pad,), jnp.int32)]).reshape(16, _NCHUNK, _CHUNK)
    row = jnp.concatenate([edge_index[0].astype(jnp.int32),
                           jnp.zeros((pad,), jnp.int32)]).reshape(16, _NCHUNK, _CHUNK)
    w = jnp.concatenate([edge_weight,
                         jnp.zeros((pad,), jnp.float32)]).reshape(16, _EPT)

    x_pad = jnp.concatenate([x, jnp.zeros((_NP - _N, x.shape[1]), jnp.float32)])
    h = _mlp(x_pad, W1, b1, W2, b2)

    spmm = _make_spmm()

    t = spmm(h, col, row, w)
    fp01 = jnp.stack([fp[0], fp[1]]).reshape(1, 2)
    p_prev2 = h
    p_prev1, y = _combine(True, t, h, h, fp01)

    for i in range(2, _K + 1):
        t = spmm(p_prev1, col, row, w)
        p_next, y = _combine(False, t, p_prev2, y, fp[i].reshape(1, 1))
        p_prev2, p_prev1 = p_prev1, p_next

    return _logsoftmax(y)


# speedup vs baseline: 1.0029x; 1.0029x over previous
import functools
import math

import jax
import jax.numpy as jnp
import numpy as np
from jax import lax
from jax.experimental import pallas as pl
from jax.experimental.pallas import tpu as pltpu
from jax.experimental.pallas import tpu_sc as plsc

_K = 10
_N = 10000
_E = 160000
_C = 128
_F2 = _C // 2

_NP = 10240
_CHUNK = 128
_NCHUNK = 81
_EPT = _CHUNK * _NCHUNK
_EPAD = _EPT * 16
_RPT = _NP // 16
_NB = 3


def _cheby_cols():
    k = _K
    nodes = [math.cos((k - j + 0.5) * math.pi / (k + 1)) for j in range(k + 1)]
    cols = []
    for xj in nodes:
        vals = [0.0] * (k + 1)
        for j in range(k + 1):
            if j == 0:
                vals[j] = 1.0
            elif j == 1:
                vals[j] = xj
            else:
                vals[j] = 2.0 * xj * vals[j - 1] - vals[j - 2]
        cols.append(vals)
    return np.array(cols, dtype=np.float32).T


_CV = _cheby_cols()


def _mlp_body(x_ref, w1_ref, b1_ref, w2a_ref, w2b_ref, b2_ref, o_ref):
    h = jnp.dot(x_ref[...], w1_ref[...], preferred_element_type=jnp.float32)
    h = jnp.maximum(h + b1_ref[...], 0.0)
    oa = jnp.dot(h, w2a_ref[...], preferred_element_type=jnp.float32)
    ob = jnp.dot(h, w2b_ref[...], preferred_element_type=jnp.float32)
    o_ref[0] = oa + b2_ref[0, 0:1, :]
    o_ref[1] = ob + b2_ref[0, 1:2, :]


def _mlp(x_pad, W1, b1, W2, b2):
    blk = 2048
    grid = (_NP // blk,)
    return pl.pallas_call(
        _mlp_body,
        grid=grid,
        in_specs=[
            pl.BlockSpec((blk, 256), lambda i: (i, 0)),
            pl.BlockSpec((256, 512), lambda i: (0, 0)),
            pl.BlockSpec((1, 512), lambda i: (0, 0)),
            pl.BlockSpec((512, _F2), lambda i: (0, 0)),
            pl.BlockSpec((512, _F2), lambda i: (0, 0)),
            pl.BlockSpec((1, 2, _F2), lambda i: (0, 0, 0)),
        ],
        out_specs=pl.BlockSpec((2, blk, _F2), lambda i: (0, i, 0)),
        out_shape=jax.ShapeDtypeStruct((2, _NP, _F2), jnp.float32),
    )(x_pad, W1, b1.reshape(1, 512), W2[:, :_F2], W2[:, _F2:],
      b2.reshape(1, 2, _F2))


def _spmm_sc_body(p_hbm, col_hbm, row_hbm, w_hbm, out_hbm,
                  colm, rowm, wm, rows_v, sbuf, accum_sh, sem_g, sem_s):
    c = lax.axis_index("c")
    s = lax.axis_index("s")

    pltpu.sync_copy(col_hbm.at[s], colm)
    pltpu.sync_copy(row_hbm.at[s], rowm)
    pltpu.sync_copy(w_hbm.at[s], wm)

    def _zero_body(r, carry):
        for f in range(_F2 // 16):
            for b in range(_NB):
                sbuf[b][r, pl.ds(f * 16, 16)] = jnp.zeros((16,), jnp.float32)
        return carry
    lax.fori_loop(0, _CHUNK, _zero_body, 0)
    for t in range(_RPT // _CHUNK):
        pltpu.sync_copy(sbuf[0], accum_sh.at[pl.ds(s * _RPT + t * _CHUNK, _CHUNK)])
    plsc.subcore_barrier()

    for b in range(_NB):
        pltpu.async_copy(p_hbm.at[c].at[colm.at[b]], rows_v[b], sem_g[b])
        pltpu.async_copy(sbuf[b], accum_sh.at[rowm.at[b]], sem_s[b], add=True)

    def _outer(jj, carry):
        for b in range(_NB):
            j = jj * _NB + b
            pltpu.make_async_copy(p_hbm.at[c].at[colm.at[j]], rows_v[b],
                                  sem_g[b]).wait()
            pltpu.make_async_copy(sbuf[b], accum_sh.at[rowm.at[j]],
                                  sem_s[b]).wait()

            def _mul_body(r, carry2):
                idx = jnp.full((16,), r, jnp.int32)
                wv = plsc.load_gather(wm, [jnp.full((16,), j, jnp.int32), idx])
                for f in range(_F2 // 16):
                    sl = pl.ds(f * 16, 16)
                    sbuf[b][r, sl] = rows_v[b][r, sl] * wv
                return carry2
            lax.fori_loop(0, _CHUNK, _mul_body, 0, unroll=4)

            pltpu.async_copy(sbuf[b], accum_sh.at[rowm.at[j]], sem_s[b],
                             add=True)

            @pl.when(j + _NB < _NCHUNK)
            def _():
                pltpu.async_copy(p_hbm.at[c].at[colm.at[j + _NB]], rows_v[b],
                                 sem_g[b])
        return carry
    lax.fori_loop(0, _NCHUNK // _NB, _outer, 0)

    for b in range(_NB):
        pltpu.make_async_copy(sbuf[b], accum_sh.at[rowm.at[b]], sem_s[b]).wait()

    plsc.subcore_barrier()

    sl = pl.ds(s * _RPT, _RPT)
    pltpu.sync_copy(accum_sh.at[sl], out_hbm.at[c].at[sl])


def _make_spmm():
    mesh = plsc.VectorSubcoreMesh(core_axis_name="c", subcore_axis_name="s")
    return pl.kernel(
        _spmm_sc_body,
        out_type=jax.ShapeDtypeStruct((2, _NP, _F2), jnp.float32),
        mesh=mesh,
        scratch_types=[
            pltpu.VMEM((_NCHUNK, _CHUNK), jnp.int32),
            pltpu.VMEM((_NCHUNK, _CHUNK), jnp.int32),
            pltpu.VMEM((_NCHUNK, _CHUNK), jnp.float32),
            [pltpu.VMEM((_CHUNK, _F2), jnp.float32) for _ in range(_NB)],
            [pltpu.VMEM((_CHUNK, _F2), jnp.float32) for _ in range(_NB)],
            pltpu.VMEM_SHARED((_NP, _F2), jnp.float32),
            [pltpu.SemaphoreType.DMA for _ in range(_NB)],
            [pltpu.SemaphoreType.DMA for _ in range(_NB)],
        ],
        compiler_params=pltpu.CompilerParams(use_tc_tiling_on_sc=False,
                                             needs_layout_passes=False),
    )


def _combine_body(first, t_ref, prev_ref, y_ref, fp_ref, p_ref, yo_ref):
    if first:
        p = t_ref[...]
        yo_ref[...] = fp_ref[0, 0] * prev_ref[...] + fp_ref[0, 1] * p
    else:
        p = 2.0 * t_ref[...] - prev_ref[...]
        yo_ref[...] = y_ref[...] + fp_ref[0, 0] * p
    p_ref[...] = p


def _combine(first, t, prev, y, fp_sc):
    blk = 2048
    grid = (_NP // blk,)
    spec = pl.BlockSpec((2, blk, _F2), lambda i: (0, i, 0))
    return pl.pallas_call(
        functools.partial(_combine_body, first),
        grid=grid,
        in_specs=[spec, spec, spec, pl.BlockSpec(memory_space=pltpu.SMEM)],
        out_specs=[spec, spec],
        out_shape=[
            jax.ShapeDtypeStruct((2, _NP, _F2), jnp.float32),
            jax.ShapeDtypeStruct((2, _NP, _F2), jnp.float32),
        ],
    )(t, prev, y, fp_sc)


def _logsoftmax_body(y_ref, o_ref):
    y = jnp.concatenate([y_ref[0], y_ref[1]], axis=1)
    m = jnp.max(y, axis=1, keepdims=True)
    e = jnp.exp(y - m)
    s = jnp.sum(e, axis=1, keepdims=True)
    o_ref[...] = y - m - jnp.log(s)


def _logsoftmax(y):
    blk = 2000
    return pl.pallas_call(
        _logsoftmax_body,
        grid=(_N // blk,),
        in_specs=[pl.BlockSpec((2, blk, _F2), lambda i: (0, i, 0))],
        out_specs=pl.BlockSpec((blk, _C), lambda i: (i, 0)),
        out_shape=jax.ShapeDtypeStruct((_N, _C), jnp.float32),
    )(y)


def kernel(x, edge_index, edge_weight, W1, b1, W2, b2, filter_param):
    fp = jnp.maximum(filter_param, 0.0)
    fp = jnp.asarray(_CV) @ fp
    fp = 2.0 * fp / (_K + 1)
    fp = fp.at[0].set(fp[0] / 2.0)
    signs = jnp.asarray(np.where(np.arange(_K + 1) % 2 == 0, 1.0, -1.0)
                        .astype(np.float32))
    fp = fp[:, 0] * signs

    pad = _EPAD - _E
    col = jnp.concatenate([edge_index[1].astype(jnp.int32),
                           jnp.zeros((pad,), jnp.int32)]).reshape(16, _NCHUNK, _CHUNK)
    row = jnp.concatenate([edge_index[0].astype(jnp.int32),
                           jnp.zeros((pad,), jnp.int32)]).reshape(16, _NCHUNK, _CHUNK)
    w = jnp.concatenate([edge_weight,
                         jnp.zeros((pad,), jnp.float32)]).reshape(16, _NCHUNK, _CHUNK)

    x_pad = jnp.concatenate([x, jnp.zeros((_NP - _N, x.shape[1]), jnp.float32)])
    h = _mlp(x_pad, W1, b1, W2, b2)

    spmm = _make_spmm()

    t = spmm(h, col, row, w)
    fp01 = jnp.stack([fp[0], fp[1]]).reshape(1, 2)
    p_prev2 = h
    p_prev1, y = _combine(True, t, h, h, fp01)

    for i in range(2, _K + 1):
        t = spmm(p_prev1, col, row, w)
        p_next, y = _combine(False, t, p_prev2, y, fp[i].reshape(1, 1))
        p_prev2, p_prev1 = p_prev1, p_next

    return _logsoftmax(y)


# speedup vs baseline: 1.0701x; 1.0670x over previous
import functools
import math

import jax
import jax.numpy as jnp
import numpy as np
from jax import lax
from jax.experimental import pallas as pl
from jax.experimental.pallas import tpu as pltpu
from jax.experimental.pallas import tpu_sc as plsc

_K = 10
_N = 10000
_E = 160000
_C = 128
_F2 = _C // 2

_NP = 10240
_CHUNK = 128
_NCHUNK = 80
_EPT = _CHUNK * _NCHUNK
_EPAD = _EPT * 16
_RPT = _NP // 16


def _cheby_cols():
    k = _K
    nodes = [math.cos((k - j + 0.5) * math.pi / (k + 1)) for j in range(k + 1)]
    cols = []
    for xj in nodes:
        vals = [0.0] * (k + 1)
        for j in range(k + 1):
            if j == 0:
                vals[j] = 1.0
            elif j == 1:
                vals[j] = xj
            else:
                vals[j] = 2.0 * xj * vals[j - 1] - vals[j - 2]
        cols.append(vals)
    return np.array(cols, dtype=np.float32).T


_CV = _cheby_cols()


def _mlp_body(x_ref, w1_ref, b1_ref, w2a_ref, w2b_ref, b2_ref, o_ref):
    h = jnp.dot(x_ref[...], w1_ref[...], preferred_element_type=jnp.float32)
    h = jnp.maximum(h + b1_ref[...], 0.0)
    oa = jnp.dot(h, w2a_ref[...], preferred_element_type=jnp.float32)
    ob = jnp.dot(h, w2b_ref[...], preferred_element_type=jnp.float32)
    o_ref[0] = oa + b2_ref[0, 0:1, :]
    o_ref[1] = ob + b2_ref[0, 1:2, :]


def _mlp(x_pad, W1, b1, W2, b2):
    blk = 2048
    grid = (_NP // blk,)
    return pl.pallas_call(
        _mlp_body,
        grid=grid,
        in_specs=[
            pl.BlockSpec((blk, 256), lambda i: (i, 0)),
            pl.BlockSpec((256, 512), lambda i: (0, 0)),
            pl.BlockSpec((1, 512), lambda i: (0, 0)),
            pl.BlockSpec((512, _F2), lambda i: (0, 0)),
            pl.BlockSpec((512, _F2), lambda i: (0, 0)),
            pl.BlockSpec((1, 2, _F2), lambda i: (0, 0, 0)),
        ],
        out_specs=pl.BlockSpec((2, blk, _F2), lambda i: (0, i, 0)),
        out_shape=jax.ShapeDtypeStruct((2, _NP, _F2), jnp.float32),
    )(x_pad, W1, b1.reshape(1, 512), W2[:, :_F2], W2[:, _F2:],
      b2.reshape(1, 2, _F2))


def _spmm_sc_body(p_hbm, col_hbm, row_hbm, w_hbm, out_hbm,
                  colm, rowm, wm, rows_v, sbuf, accum_sh, sem_g, sem_s):
    c = lax.axis_index("c")
    s = lax.axis_index("s")

    pltpu.sync_copy(col_hbm.at[s], colm)
    pltpu.sync_copy(row_hbm.at[s], rowm)
    pltpu.sync_copy(w_hbm.at[s], wm)

    def _zero_body(r, carry):
        for f in range(_F2 // 16):
            sbuf[0][r, pl.ds(f * 16, 16)] = jnp.zeros((16,), jnp.float32)
            sbuf[1][r, pl.ds(f * 16, 16)] = jnp.zeros((16,), jnp.float32)
        return carry
    lax.fori_loop(0, _CHUNK, _zero_body, 0)
    for t in range(_RPT // _CHUNK):
        pltpu.sync_copy(sbuf[0], accum_sh.at[pl.ds(s * _RPT + t * _CHUNK, _CHUNK)])
    plsc.subcore_barrier()

    for b in range(2):
        pltpu.async_copy(p_hbm.at[c].at[colm.at[b]], rows_v[b], sem_g[b])
        pltpu.async_copy(sbuf[b], accum_sh.at[rowm.at[b]], sem_s[b], add=True)

    def _outer(jj, carry):
        for b in range(2):
            j = jj * 2 + b
            pltpu.make_async_copy(p_hbm.at[c].at[colm.at[j]], rows_v[b],
                                  sem_g[b]).wait()
            pltpu.make_async_copy(sbuf[b], accum_sh.at[rowm.at[j]],
                                  sem_s[b]).wait()

            def _mul_body(r, carry2):
                idx = jnp.full((16,), r, jnp.int32)
                wv = plsc.load_gather(wm, [jnp.full((16,), j, jnp.int32), idx])
                for f in range(_F2 // 16):
                    sl = pl.ds(f * 16, 16)
                    sbuf[b][r, sl] = rows_v[b][r, sl] * wv
                return carry2
            lax.fori_loop(0, _CHUNK, _mul_body, 0, unroll=8)

            pltpu.async_copy(sbuf[b], accum_sh.at[rowm.at[j]], sem_s[b],
                             add=True)

            @pl.when(j + 2 < _NCHUNK)
            def _():
                pltpu.async_copy(p_hbm.at[c].at[colm.at[j + 2]], rows_v[b],
                                 sem_g[b])
        return carry
    lax.fori_loop(0, _NCHUNK // 2, _outer, 0)

    for b in range(2):
        pltpu.make_async_copy(sbuf[b], accum_sh.at[rowm.at[b]], sem_s[b]).wait()

    plsc.subcore_barrier()

    sl = pl.ds(s * _RPT, _RPT)
    pltpu.sync_copy(accum_sh.at[sl], out_hbm.at[c].at[sl])


def _make_spmm():
    mesh = plsc.VectorSubcoreMesh(core_axis_name="c", subcore_axis_name="s")
    return pl.kernel(
        _spmm_sc_body,
        out_type=jax.ShapeDtypeStruct((2, _NP, _F2), jnp.float32),
        mesh=mesh,
        scratch_types=[
            pltpu.VMEM((_NCHUNK, _CHUNK), jnp.int32),
            pltpu.VMEM((_NCHUNK, _CHUNK), jnp.int32),
            pltpu.VMEM((_NCHUNK, _CHUNK), jnp.float32),
            [pltpu.VMEM((_CHUNK, _F2), jnp.float32) for _ in range(2)],
            [pltpu.VMEM((_CHUNK, _F2), jnp.float32) for _ in range(2)],
            pltpu.VMEM_SHARED((_NP, _F2), jnp.float32),
            [pltpu.SemaphoreType.DMA for _ in range(2)],
            [pltpu.SemaphoreType.DMA for _ in range(2)],
        ],
        compiler_params=pltpu.CompilerParams(use_tc_tiling_on_sc=False,
                                             needs_layout_passes=False),
    )


def _combine_body(first, t_ref, prev_ref, y_ref, fp_ref, p_ref, yo_ref):
    if first:
        p = t_ref[...]
        yo_ref[...] = fp_ref[0, 0] * prev_ref[...] + fp_ref[0, 1] * p
    else:
        p = 2.0 * t_ref[...] - prev_ref[...]
        yo_ref[...] = y_ref[...] + fp_ref[0, 0] * p
    p_ref[...] = p


def _combine(first, t, prev, y, fp_sc):
    blk = 2048
    grid = (_NP // blk,)
    spec = pl.BlockSpec((2, blk, _F2), lambda i: (0, i, 0))
    return pl.pallas_call(
        functools.partial(_combine_body, first),
        grid=grid,
        in_specs=[spec, spec, spec, pl.BlockSpec(memory_space=pltpu.SMEM)],
        out_specs=[spec, spec],
        out_shape=[
            jax.ShapeDtypeStruct((2, _NP, _F2), jnp.float32),
            jax.ShapeDtypeStruct((2, _NP, _F2), jnp.float32),
        ],
    )(t, prev, y, fp_sc)


def _logsoftmax_body(y_ref, o_ref):
    y = jnp.concatenate([y_ref[0], y_ref[1]], axis=1)
    m = jnp.max(y, axis=1, keepdims=True)
    e = jnp.exp(y - m)
    s = jnp.sum(e, axis=1, keepdims=True)
    o_ref[...] = y - m - jnp.log(s)


def _logsoftmax(y):
    blk = 2000
    return pl.pallas_call(
        _logsoftmax_body,
        grid=(_N // blk,),
        in_specs=[pl.BlockSpec((2, blk, _F2), lambda i: (0, i, 0))],
        out_specs=pl.BlockSpec((blk, _C), lambda i: (i, 0)),
        out_shape=jax.ShapeDtypeStruct((_N, _C), jnp.float32),
    )(y)


def kernel(x, edge_index, edge_weight, W1, b1, W2, b2, filter_param):
    fp = jnp.maximum(filter_param, 0.0)
    fp = jnp.asarray(_CV) @ fp
    fp = 2.0 * fp / (_K + 1)
    fp = fp.at[0].set(fp[0] / 2.0)
    signs = jnp.asarray(np.where(np.arange(_K + 1) % 2 == 0, 1.0, -1.0)
                        .astype(np.float32))
    fp = fp[:, 0] * signs

    pad = _EPAD - _E
    col = jnp.concatenate([edge_index[1].astype(jnp.int32),
                           jnp.zeros((pad,), jnp.int32)]).reshape(16, _NCHUNK, _CHUNK)
    row = jnp.concatenate([edge_index[0].astype(jnp.int32),
                           jnp.zeros((pad,), jnp.int32)]).reshape(16, _NCHUNK, _CHUNK)
    w = jnp.concatenate([edge_weight,
                         jnp.zeros((pad,), jnp.float32)]).reshape(16, _NCHUNK, _CHUNK)

    x_pad = jnp.concatenate([x, jnp.zeros((_NP - _N, x.shape[1]), jnp.float32)])
    h = _mlp(x_pad, W1, b1, W2, b2)

    spmm = _make_spmm()

    t = spmm(h, col, row, w)
    fp01 = jnp.stack([fp[0], fp[1]]).reshape(1, 2)
    p_prev2 = h
    p_prev1, y = _combine(True, t, h, h, fp01)

    for i in range(2, _K + 1):
        t = spmm(p_prev1, col, row, w)
        p_next, y = _combine(False, t, p_prev2, y, fp[i].reshape(1, 1))
        p_prev2, p_prev1 = p_prev1, p_next

    return _logsoftmax(y)


# speedup vs baseline: 1.0815x; 1.0107x over previous
import functools
import math

import jax
import jax.numpy as jnp
import numpy as np
from jax import lax
from jax.experimental import pallas as pl
from jax.experimental.pallas import tpu as pltpu
from jax.experimental.pallas import tpu_sc as plsc

_K = 10
_N = 10000
_E = 160000
_C = 128
_F2 = _C // 2

_NP = 10240
_CHUNK = 128
_NCHUNK = 80
_EPT = _CHUNK * _NCHUNK
_EPAD = _EPT * 16
_RPT = _NP // 16


def _cheby_cols():
    k = _K
    nodes = [math.cos((k - j + 0.5) * math.pi / (k + 1)) for j in range(k + 1)]
    cols = []
    for xj in nodes:
        vals = [0.0] * (k + 1)
        for j in range(k + 1):
            if j == 0:
                vals[j] = 1.0
            elif j == 1:
                vals[j] = xj
            else:
                vals[j] = 2.0 * xj * vals[j - 1] - vals[j - 2]
        cols.append(vals)
    return np.array(cols, dtype=np.float32).T


_CV = _cheby_cols()


def _mlp_body(x_ref, w1_ref, b1_ref, w2a_ref, w2b_ref, b2_ref, o_ref):
    h = jnp.dot(x_ref[...], w1_ref[...], preferred_element_type=jnp.float32)
    h = jnp.maximum(h + b1_ref[...], 0.0)
    oa = jnp.dot(h, w2a_ref[...], preferred_element_type=jnp.float32)
    ob = jnp.dot(h, w2b_ref[...], preferred_element_type=jnp.float32)
    o_ref[0] = oa + b2_ref[0, 0:1, :]
    o_ref[1] = ob + b2_ref[0, 1:2, :]


def _mlp(x_pad, W1, b1, W2, b2):
    blk = 2048
    grid = (_NP // blk,)
    return pl.pallas_call(
        _mlp_body,
        grid=grid,
        in_specs=[
            pl.BlockSpec((blk, 256), lambda i: (i, 0)),
            pl.BlockSpec((256, 512), lambda i: (0, 0)),
            pl.BlockSpec((1, 512), lambda i: (0, 0)),
            pl.BlockSpec((512, _F2), lambda i: (0, 0)),
            pl.BlockSpec((512, _F2), lambda i: (0, 0)),
            pl.BlockSpec((1, 2, _F2), lambda i: (0, 0, 0)),
        ],
        out_specs=pl.BlockSpec((2, blk, _F2), lambda i: (0, i, 0)),
        out_shape=jax.ShapeDtypeStruct((2, _NP, _F2), jnp.float32),
    )(x_pad, W1, b1.reshape(1, 512), W2[:, :_F2], W2[:, _F2:],
      b2.reshape(1, 2, _F2))


def _spmm_sc_body(p_hbm, col_hbm, row_hbm, w_hbm, out_hbm,
                  colm, rowm, wm, rows_v, sbuf, accum_sh, sem_g, sem_s):
    c = lax.axis_index("c")
    s = lax.axis_index("s")

    pltpu.sync_copy(col_hbm.at[s], colm)
    pltpu.sync_copy(row_hbm.at[s], rowm)
    pltpu.sync_copy(w_hbm.at[s], wm)

    def _zero_body(r, carry):
        for f in range(_F2 // 16):
            sbuf[0][r, pl.ds(f * 16, 16)] = jnp.zeros((16,), jnp.float32)
            sbuf[1][r, pl.ds(f * 16, 16)] = jnp.zeros((16,), jnp.float32)
        return carry
    lax.fori_loop(0, _CHUNK, _zero_body, 0)
    for t in range(_RPT // _CHUNK):
        pltpu.sync_copy(sbuf[0], accum_sh.at[pl.ds(s * _RPT + t * _CHUNK, _CHUNK)])
    plsc.subcore_barrier()

    for b in range(2):
        pltpu.async_copy(p_hbm.at[c].at[colm.at[b]], rows_v[b], sem_g[b])
        pltpu.async_copy(sbuf[b], accum_sh.at[rowm.at[b]], sem_s[b], add=True)

    def _outer(jj, carry):
        for t in range(4):
            j = jj * 4 + t
            b4 = t
            b2 = t % 2
            pltpu.make_async_copy(p_hbm.at[c].at[colm.at[j]], rows_v[b4],
                                  sem_g[b4]).wait()

            @pl.when(j + 2 < _NCHUNK)
            def _():
                pltpu.async_copy(p_hbm.at[c].at[colm.at[j + 2]],
                                 rows_v[(t + 2) % 4], sem_g[(t + 2) % 4])

            pltpu.make_async_copy(sbuf[b2], accum_sh.at[rowm.at[j]],
                                  sem_s[b2]).wait()

            def _mul_body(r, carry2):
                idx = jnp.full((16,), r, jnp.int32)
                wv = plsc.load_gather(wm, [jnp.full((16,), j, jnp.int32), idx])
                for f in range(_F2 // 16):
                    sl = pl.ds(f * 16, 16)
                    sbuf[b2][r, sl] = rows_v[b4][r, sl] * wv
                return carry2
            lax.fori_loop(0, _CHUNK, _mul_body, 0, unroll=4)

            pltpu.async_copy(sbuf[b2], accum_sh.at[rowm.at[j]], sem_s[b2],
                             add=True)
        return carry
    lax.fori_loop(0, _NCHUNK // 4, _outer, 0)

    for b in range(2):
        pltpu.make_async_copy(sbuf[b], accum_sh.at[rowm.at[b]], sem_s[b]).wait()

    plsc.subcore_barrier()

    sl = pl.ds(s * _RPT, _RPT)
    pltpu.sync_copy(accum_sh.at[sl], out_hbm.at[c].at[sl])


def _make_spmm():
    mesh = plsc.VectorSubcoreMesh(core_axis_name="c", subcore_axis_name="s")
    return pl.kernel(
        _spmm_sc_body,
        out_type=jax.ShapeDtypeStruct((2, _NP, _F2), jnp.float32),
        mesh=mesh,
        scratch_types=[
            pltpu.VMEM((_NCHUNK, _CHUNK), jnp.int32),
            pltpu.VMEM((_NCHUNK, _CHUNK), jnp.int32),
            pltpu.VMEM((_NCHUNK, _CHUNK), jnp.float32),
            [pltpu.VMEM((_CHUNK, _F2), jnp.float32) for _ in range(4)],
            [pltpu.VMEM((_CHUNK, _F2), jnp.float32) for _ in range(2)],
            pltpu.VMEM_SHARED((_NP, _F2), jnp.float32),
            [pltpu.SemaphoreType.DMA for _ in range(4)],
            [pltpu.SemaphoreType.DMA for _ in range(2)],
        ],
        compiler_params=pltpu.CompilerParams(use_tc_tiling_on_sc=False,
                                             needs_layout_passes=False),
    )


def _combine_body(first, t_ref, prev_ref, y_ref, fp_ref, p_ref, yo_ref):
    if first:
        p = t_ref[...]
        yo_ref[...] = fp_ref[0, 0] * prev_ref[...] + fp_ref[0, 1] * p
    else:
        p = 2.0 * t_ref[...] - prev_ref[...]
        yo_ref[...] = y_ref[...] + fp_ref[0, 0] * p
    p_ref[...] = p


def _combine(first, t, prev, y, fp_sc):
    blk = 2048
    grid = (_NP // blk,)
    spec = pl.BlockSpec((2, blk, _F2), lambda i: (0, i, 0))
    return pl.pallas_call(
        functools.partial(_combine_body, first),
        grid=grid,
        in_specs=[spec, spec, spec, pl.BlockSpec(memory_space=pltpu.SMEM)],
        out_specs=[spec, spec],
        out_shape=[
            jax.ShapeDtypeStruct((2, _NP, _F2), jnp.float32),
            jax.ShapeDtypeStruct((2, _NP, _F2), jnp.float32),
        ],
    )(t, prev, y, fp_sc)


def _logsoftmax_body(y_ref, o_ref):
    y = jnp.concatenate([y_ref[0], y_ref[1]], axis=1)
    m = jnp.max(y, axis=1, keepdims=True)
    e = jnp.exp(y - m)
    s = jnp.sum(e, axis=1, keepdims=True)
    o_ref[...] = y - m - jnp.log(s)


def _logsoftmax(y):
    blk = 2000
    return pl.pallas_call(
        _logsoftmax_body,
        grid=(_N // blk,),
        in_specs=[pl.BlockSpec((2, blk, _F2), lambda i: (0, i, 0))],
        out_specs=pl.BlockSpec((blk, _C), lambda i: (i, 0)),
        out_shape=jax.ShapeDtypeStruct((_N, _C), jnp.float32),
    )(y)


def kernel(x, edge_index, edge_weight, W1, b1, W2, b2, filter_param):
    fp = jnp.maximum(filter_param, 0.0)
    fp = jnp.asarray(_CV) @ fp
    fp = 2.0 * fp / (_K + 1)
    fp = fp.at[0].set(fp[0] / 2.0)
    signs = jnp.asarray(np.where(np.arange(_K + 1) % 2 == 0, 1.0, -1.0)
                        .astype(np.float32))
    fp = fp[:, 0] * signs

    pad = _EPAD - _E
    col = jnp.concatenate([edge_index[1].astype(jnp.int32),
                           jnp.zeros((pad,), jnp.int32)]).reshape(16, _NCHUNK, _CHUNK)
    row = jnp.concatenate([edge_index[0].astype(jnp.int32),
                           jnp.zeros((pad,), jnp.int32)]).reshape(16, _NCHUNK, _CHUNK)
    w = jnp.concatenate([edge_weight,
                         jnp.zeros((pad,), jnp.float32)]).reshape(16, _NCHUNK, _CHUNK)

    x_pad = jnp.concatenate([x, jnp.zeros((_NP - _N, x.shape[1]), jnp.float32)])
    h = _mlp(x_pad, W1, b1, W2, b2)

    spmm = _make_spmm()

    t = spmm(h, col, row, w)
    fp01 = jnp.stack([fp[0], fp[1]]).reshape(1, 2)
    p_prev2 = h
    p_prev1, y = _combine(True, t, h, h, fp01)

    for i in range(2, _K + 1):
        t = spmm(p_prev1, col, row, w)
        p_next, y = _combine(False, t, p_prev2, y, fp[i].reshape(1, 1))
        p_prev2, p_prev1 = p_prev1, p_next

    return _logsoftmax(y)


# speedup vs baseline: 1.0902x; 1.0080x over previous
import functools
import math

import jax
import jax.numpy as jnp
import numpy as np
from jax import lax
from jax.experimental import pallas as pl
from jax.experimental.pallas import tpu as pltpu
from jax.experimental.pallas import tpu_sc as plsc

_K = 10
_N = 10000
_E = 160000
_C = 128
_F2 = _C // 2

_NP = 10240
_CHUNK = 128
_NCHUNK = 80
_EPT = _CHUNK * _NCHUNK
_EPAD = _EPT * 16
_RPT = _NP // 16


def _cheby_cols():
    k = _K
    nodes = [math.cos((k - j + 0.5) * math.pi / (k + 1)) for j in range(k + 1)]
    cols = []
    for xj in nodes:
        vals = [0.0] * (k + 1)
        for j in range(k + 1):
            if j == 0:
                vals[j] = 1.0
            elif j == 1:
                vals[j] = xj
            else:
                vals[j] = 2.0 * xj * vals[j - 1] - vals[j - 2]
        cols.append(vals)
    return np.array(cols, dtype=np.float32).T


_CV = _cheby_cols()


def _mlp_body(x_ref, w1_ref, b1_ref, w2a_ref, w2b_ref, b2_ref, o_ref):
    h = jnp.dot(x_ref[...], w1_ref[...], preferred_element_type=jnp.float32)
    h = jnp.maximum(h + b1_ref[...], 0.0)
    oa = jnp.dot(h, w2a_ref[...], preferred_element_type=jnp.float32)
    ob = jnp.dot(h, w2b_ref[...], preferred_element_type=jnp.float32)
    o_ref[0] = oa + b2_ref[0, 0:1, :]
    o_ref[1] = ob + b2_ref[0, 1:2, :]


def _mlp(x_pad, W1, b1, W2, b2):
    blk = 2048
    grid = (_NP // blk,)
    return pl.pallas_call(
        _mlp_body,
        grid=grid,
        in_specs=[
            pl.BlockSpec((blk, 256), lambda i: (i, 0)),
            pl.BlockSpec((256, 512), lambda i: (0, 0)),
            pl.BlockSpec((1, 512), lambda i: (0, 0)),
            pl.BlockSpec((512, _F2), lambda i: (0, 0)),
            pl.BlockSpec((512, _F2), lambda i: (0, 0)),
            pl.BlockSpec((1, 2, _F2), lambda i: (0, 0, 0)),
        ],
        out_specs=pl.BlockSpec((2, blk, _F2), lambda i: (0, i, 0)),
        out_shape=jax.ShapeDtypeStruct((2, _NP, _F2), jnp.float32),
    )(x_pad, W1, b1.reshape(1, 512), W2[:, :_F2], W2[:, _F2:],
      b2.reshape(1, 2, _F2))


def _spmm_sc_body(p_hbm, col_hbm, row_hbm, w_hbm, out_hbm,
                  colm, rowm, wm, rows_v, sbuf, accum_sh, sem_g, sem_s):
    c = lax.axis_index("c")
    s = lax.axis_index("s")

    pltpu.sync_copy(col_hbm.at[s], colm)
    pltpu.sync_copy(row_hbm.at[s], rowm)
    pltpu.sync_copy(w_hbm.at[s], wm)

    def _zero_body(r, carry):
        for f in range(_F2 // 16):
            sbuf[0][r, pl.ds(f * 16, 16)] = jnp.zeros((16,), jnp.float32)
            sbuf[1][r, pl.ds(f * 16, 16)] = jnp.zeros((16,), jnp.float32)
        return carry
    lax.fori_loop(0, _CHUNK, _zero_body, 0)
    for t in range(_RPT // _CHUNK):
        pltpu.sync_copy(sbuf[0], accum_sh.at[pl.ds(s * _RPT + t * _CHUNK, _CHUNK)])
    plsc.subcore_barrier()

    for b in range(3):
        pltpu.async_copy(p_hbm.at[c].at[colm.at[b]], rows_v[b], sem_g[b])
    for b in range(2):
        pltpu.async_copy(sbuf[b], accum_sh.at[rowm.at[b]], sem_s[b], add=True)

    def _outer(jj, carry):
        for t in range(4):
            j = jj * 4 + t
            b4 = t
            b2 = t % 2
            pltpu.make_async_copy(p_hbm.at[c].at[colm.at[j]], rows_v[b4],
                                  sem_g[b4]).wait()

            @pl.when(j + 3 < _NCHUNK)
            def _():
                pltpu.async_copy(p_hbm.at[c].at[colm.at[j + 3]],
                                 rows_v[(t + 3) % 4], sem_g[(t + 3) % 4])

            pltpu.make_async_copy(sbuf[b2], accum_sh.at[rowm.at[j]],
                                  sem_s[b2]).wait()

            def _mul_body(r, carry2):
                idx = jnp.full((16,), r, jnp.int32)
                wv = plsc.load_gather(wm, [jnp.full((16,), j, jnp.int32), idx])
                for f in range(_F2 // 16):
                    sl = pl.ds(f * 16, 16)
                    sbuf[b2][r, sl] = rows_v[b4][r, sl] * wv
                return carry2
            lax.fori_loop(0, _CHUNK, _mul_body, 0, unroll=4)

            pltpu.async_copy(sbuf[b2], accum_sh.at[rowm.at[j]], sem_s[b2],
                             add=True)
        return carry
    lax.fori_loop(0, _NCHUNK // 4, _outer, 0)

    for b in range(2):
        pltpu.make_async_copy(sbuf[b], accum_sh.at[rowm.at[b]], sem_s[b]).wait()

    plsc.subcore_barrier()

    sl = pl.ds(s * _RPT, _RPT)
    pltpu.sync_copy(accum_sh.at[sl], out_hbm.at[c].at[sl])


def _make_spmm():
    mesh = plsc.VectorSubcoreMesh(core_axis_name="c", subcore_axis_name="s")
    return pl.kernel(
        _spmm_sc_body,
        out_type=jax.ShapeDtypeStruct((2, _NP, _F2), jnp.float32),
        mesh=mesh,
        scratch_types=[
            pltpu.VMEM((_NCHUNK, _CHUNK), jnp.int32),
            pltpu.VMEM((_NCHUNK, _CHUNK), jnp.int32),
            pltpu.VMEM((_NCHUNK, _CHUNK), jnp.float32),
            [pltpu.VMEM((_CHUNK, _F2), jnp.float32) for _ in range(4)],
            [pltpu.VMEM((_CHUNK, _F2), jnp.float32) for _ in range(2)],
            pltpu.VMEM_SHARED((_NP, _F2), jnp.float32),
            [pltpu.SemaphoreType.DMA for _ in range(4)],
            [pltpu.SemaphoreType.DMA for _ in range(2)],
        ],
        compiler_params=pltpu.CompilerParams(use_tc_tiling_on_sc=False,
                                             needs_layout_passes=False),
    )


def _combine_body(first, t_ref, prev_ref, y_ref, fp_ref, p_ref, yo_ref):
    if first:
        p = t_ref[...]
        yo_ref[...] = fp_ref[0, 0] * prev_ref[...] + fp_ref[0, 1] * p
    else:
        p = 2.0 * t_ref[...] - prev_ref[...]
        yo_ref[...] = y_ref[...] + fp_ref[0, 0] * p
    p_ref[...] = p


def _combine(first, t, prev, y, fp_sc):
    blk = 2048
    grid = (_NP // blk,)
    spec = pl.BlockSpec((2, blk, _F2), lambda i: (0, i, 0))
    return pl.pallas_call(
        functools.partial(_combine_body, first),
        grid=grid,
        in_specs=[spec, spec, spec, pl.BlockSpec(memory_space=pltpu.SMEM)],
        out_specs=[spec, spec],
        out_shape=[
            jax.ShapeDtypeStruct((2, _NP, _F2), jnp.float32),
            jax.ShapeDtypeStruct((2, _NP, _F2), jnp.float32),
        ],
    )(t, prev, y, fp_sc)


def _logsoftmax_body(y_ref, o_ref):
    y = jnp.concatenate([y_ref[0], y_ref[1]], axis=1)
    m = jnp.max(y, axis=1, keepdims=True)
    e = jnp.exp(y - m)
    s = jnp.sum(e, axis=1, keepdims=True)
    o_ref[...] = y - m - jnp.log(s)


def _logsoftmax(y):
    blk = 2000
    return pl.pallas_call(
        _logsoftmax_body,
        grid=(_N // blk,),
        in_specs=[pl.BlockSpec((2, blk, _F2), lambda i: (0, i, 0))],
        out_specs=pl.BlockSpec((blk, _C), lambda i: (i, 0)),
        out_shape=jax.ShapeDtypeStruct((_N, _C), jnp.float32),
    )(y)


def kernel(x, edge_index, edge_weight, W1, b1, W2, b2, filter_param):
    fp = jnp.maximum(filter_param, 0.0)
    fp = jnp.asarray(_CV) @ fp
    fp = 2.0 * fp / (_K + 1)
    fp = fp.at[0].set(fp[0] / 2.0)
    signs = jnp.asarray(np.where(np.arange(_K + 1) % 2 == 0, 1.0, -1.0)
                        .astype(np.float32))
    fp = fp[:, 0] * signs

    pad = _EPAD - _E
    col = jnp.concatenate([edge_index[1].astype(jnp.int32),
                           jnp.zeros((pad,), jnp.int32)]).reshape(16, _NCHUNK, _CHUNK)
    row = jnp.concatenate([edge_index[0].astype(jnp.int32),
                           jnp.zeros((pad,), jnp.int32)]).reshape(16, _NCHUNK, _CHUNK)
    w = jnp.concatenate([edge_weight,
                         jnp.zeros((pad,), jnp.float32)]).reshape(16, _NCHUNK, _CHUNK)

    x_pad = jnp.concatenate([x, jnp.zeros((_NP - _N, x.shape[1]), jnp.float32)])
    h = _mlp(x_pad, W1, b1, W2, b2)

    spmm = _make_spmm()

    t = spmm(h, col, row, w)
    fp01 = jnp.stack([fp[0], fp[1]]).reshape(1, 2)
    p_prev2 = h
    p_prev1, y = _combine(True, t, h, h, fp01)

    for i in range(2, _K + 1):
        t = spmm(p_prev1, col, row, w)
        p_next, y = _combine(False, t, p_prev2, y, fp[i].reshape(1, 1))
        p_prev2, p_prev1 = p_prev1, p_next

    return _logsoftmax(y)
